# Initial kernel scaffold; baseline (speedup 1.0000x reference)
import jax
import jax.numpy as jnp
from jax.experimental import pallas as pl


def kernel(x, edge_index, W_pool1, b_pool1, W_self1, W_neigh1, b1, W_pool2, b_pool2, W_self2, W_neigh2, b2, W_pool3, b_pool3, W_self3, W_neigh3, b3):
    raise NotImplementedError("write your pallas kernel here")


# speedup vs baseline: 1.2374x; 1.2374x over previous
import functools

import jax
import jax.numpy as jnp
from jax import lax
from jax.experimental import pallas as pl
from jax.experimental.pallas import tpu as pltpu
from jax.experimental.pallas import tpu_sc as plsc

N = 10000
E = 160000
D = 256

NW = 32
NPS = 320
NPAD = NW * NPS
ACC_ROWS = 328
SENT = 320
CHUNK = 2000
FLUSH = 2016
EB = 160768

_mesh = plsc.VectorSubcoreMesh(core_axis_name="c", subcore_axis_name="s")


def _wid():
    return lax.axis_index("s") * 2 + lax.axis_index("c")


def _bucket_body(src_ref, dst_ref, bucket_ref, counts_ref,
                 srcbuf, dstbuf, buf, cntb):
    w = _wid()
    base = w * NPS
    sent_vec = jnp.full((16,), SENT, dtype=jnp.int32)

    def chunk_body(c, total):
        pltpu.sync_copy(src_ref.at[pl.ds(c * CHUNK, CHUNK)], srcbuf)
        pltpu.sync_copy(dst_ref.at[pl.ds(c * CHUNK, CHUNK)], dstbuf)

        def scan_body(i, cnt):
            s16 = srcbuf[pl.ds(i * 16, 16)]
            d16 = dstbuf[pl.ds(i * 16, 16)]
            msk = (d16 >= base) & (d16 < base + NPS)
            pk = (s16 << 9) | (d16 - base)
            mi = jnp.where(msk, 1, 0).astype(jnp.int32)
            pos = plsc.cumsum(mi) - 1 + cnt
            plsc.store_scatter(buf, [pos], pk, mask=msk)
            npop = plsc.all_reduce_population_count(msk)
            return cnt + jnp.max(npop)

        cnt_c = lax.fori_loop(0, CHUNK // 16, scan_body, 0)
        buf[pl.ds(cnt_c, 16)] = sent_vec
        cnt8 = (cnt_c + 7) & (-8)
        off = pl.multiple_of(w * EB + total, 8)
        pltpu.sync_copy(buf, bucket_ref.at[pl.ds(off, FLUSH)])
        return total + cnt8

    total = lax.fori_loop(0, E // CHUNK, chunk_body, 0)
    buf[pl.ds(0, 16)] = sent_vec
    off = pl.multiple_of(w * EB + total, 8)
    pltpu.sync_copy(buf.at[pl.ds(0, 16)], bucket_ref.at[pl.ds(off, 16)])
    count_out = (total + 15) & (-16)
    cntb[...] = jnp.full((16,), count_out, dtype=jnp.int32)
    pltpu.sync_copy(cntb, counts_ref.at[pl.ds(pl.multiple_of(w * 16, 8), 16)])


_bucket = pl.kernel(
    _bucket_body,
    compiler_params=pltpu.CompilerParams(needs_layout_passes=False),
    out_type=(
        jax.ShapeDtypeStruct((NW * EB,), jnp.int32),
        jax.ShapeDtypeStruct((NW * 16,), jnp.int32),
    ),
    mesh=_mesh,
    scratch_types=[
        pltpu.VMEM((CHUNK,), jnp.int32),
        pltpu.VMEM((CHUNK,), jnp.int32),
        pltpu.VMEM((FLUSH,), jnp.int32),
        pltpu.VMEM((16,), jnp.int32),
    ],
)


def _segmax_body(p_ref, bucket_ref, counts_ref, out_ref,
                 pkv, srcv, rows, acc, cntb, sem):
    w = _wid()
    zeros16 = jnp.zeros((16,), dtype=jnp.float32)

    def zero_body(r, _):
        for f in range(16):
            acc[r, pl.ds(f * 16, 16)] = zeros16
        return 0

    lax.fori_loop(0, ACC_ROWS, zero_body, 0)

    pltpu.sync_copy(counts_ref.at[pl.ds(pl.multiple_of(w * 16, 8), 16)], cntb)
    count = cntb[...][0]

    def group_body(g, _):
        off = pl.multiple_of(w * EB + g * 16, 8)
        pltpu.sync_copy(bucket_ref.at[pl.ds(off, 16)], pkv)
        pk = pkv[...]
        srcv[...] = lax.shift_right_logical(pk, 9)
        dl16 = pk & 511
        pltpu.async_copy(p_ref.at[srcv], rows, sem).wait()
        for j in range(16):
            dl = dl16[j]
            for f in range(16):
                a = acc[dl, pl.ds(f * 16, 16)]
                r = rows[j, pl.ds(f * 16, 16)]
                acc[dl, pl.ds(f * 16, 16)] = jnp.maximum(a, r)
        return 0

    lax.fori_loop(0, count >> 4, group_body, 0)
    pltpu.sync_copy(acc.at[pl.ds(0, NPS)],
                    out_ref.at[pl.ds(pl.multiple_of(w * NPS, 8), NPS)])


_segmax = pl.kernel(
    _segmax_body,
    out_type=jax.ShapeDtypeStruct((NPAD, D), jnp.float32),
    mesh=_mesh,
    scratch_types=[
        pltpu.VMEM((16,), jnp.int32),
        pltpu.VMEM((16,), jnp.int32),
        pltpu.VMEM((16, D), jnp.float32),
        pltpu.VMEM((ACC_ROWS, D), jnp.float32),
        pltpu.VMEM((16,), jnp.int32),
        pltpu.SemaphoreType.DMA,
    ],
)


_RB = 1000


def _pool_body(x_ref, wp_ref, bp_ref, ws_ref, b_ref, p_ref, s_ref):
    x = x_ref[...]
    p_ref[...] = jnp.maximum(
        jnp.dot(x, wp_ref[...], preferred_element_type=jnp.float32)
        + bp_ref[...], 0.0)
    s_ref[...] = (jnp.dot(x, ws_ref[...], preferred_element_type=jnp.float32)
                  + b_ref[...])


def _pool(x, wp, bp, ws, b):
    return pl.pallas_call(
        _pool_body,
        grid=(N // _RB,),
        in_specs=[
            pl.BlockSpec((_RB, D), lambda i: (i, 0)),
            pl.BlockSpec((D, D), lambda i: (0, 0)),
            pl.BlockSpec((1, D), lambda i: (0, 0)),
            pl.BlockSpec((D, D), lambda i: (0, 0)),
            pl.BlockSpec((1, D), lambda i: (0, 0)),
        ],
        out_specs=[
            pl.BlockSpec((_RB, D), lambda i: (i, 0)),
            pl.BlockSpec((_RB, D), lambda i: (i, 0)),
        ],
        out_shape=[
            jax.ShapeDtypeStruct((N, D), jnp.float32),
            jax.ShapeDtypeStruct((N, D), jnp.float32),
        ],
    )(x, wp, bp.reshape(1, D), ws, b.reshape(1, D))


def _comb_body(s_ref, g_ref, wn_ref, o_ref, *, act):
    o = s_ref[...] + jnp.dot(g_ref[...], wn_ref[...],
                             preferred_element_type=jnp.float32)
    o_ref[...] = jnp.tanh(o) if act else o


def _comb(s, g, wn, act):
    return pl.pallas_call(
        functools.partial(_comb_body, act=act),
        grid=(N // _RB,),
        in_specs=[
            pl.BlockSpec((_RB, D), lambda i: (i, 0)),
            pl.BlockSpec((_RB, D), lambda i: (i, 0)),
            pl.BlockSpec((D, D), lambda i: (0, 0)),
        ],
        out_specs=pl.BlockSpec((_RB, D), lambda i: (i, 0)),
        out_shape=jax.ShapeDtypeStruct((N, D), jnp.float32),
    )(s, g, wn)


def kernel(x, edge_index,
           W_pool1, b_pool1, W_self1, W_neigh1, b1,
           W_pool2, b_pool2, W_self2, W_neigh2, b2,
           W_pool3, b_pool3, W_self3, W_neigh3, b3):
    bucket, counts = _bucket(edge_index[0], edge_index[1])

    def layer(h, wp, bp, ws, wn, b, act):
        p, s = _pool(h, wp, bp, ws, b)
        g = _segmax(p, bucket, counts)[:N]
        return _comb(s, g, wn, act)

    h = layer(x, W_pool1, b_pool1, W_self1, W_neigh1, b1, True)
    h = layer(h, W_pool2, b_pool2, W_self2, W_neigh2, b2, True)
    h = layer(h, W_pool3, b_pool3, W_self3, W_neigh3, b3, False)
    return h


# speedup vs baseline: 1.9026x; 1.5376x over previous
import functools

import jax
import jax.numpy as jnp
from jax import lax
from jax.experimental import pallas as pl
from jax.experimental.pallas import tpu as pltpu
from jax.experimental.pallas import tpu_sc as plsc

N = 10000
E = 160000
D = 256

NW = 32
NPS = 320
NPAD = NW * NPS
ACC_ROWS = 328
SENT = 320
CHUNK = 2000
FLUSH = 2016
EB = 160768

_mesh = plsc.VectorSubcoreMesh(core_axis_name="c", subcore_axis_name="s")


def _wid():
    return lax.axis_index("s") * 2 + lax.axis_index("c")


def _bucket_body(src_ref, dst_ref, bucket_ref, counts_ref,
                 srcbuf, dstbuf, buf, cntb):
    w = _wid()
    base = w * NPS
    sent_vec = jnp.full((16,), SENT, dtype=jnp.int32)

    def chunk_body(c, total):
        pltpu.sync_copy(src_ref.at[pl.ds(c * CHUNK, CHUNK)], srcbuf)
        pltpu.sync_copy(dst_ref.at[pl.ds(c * CHUNK, CHUNK)], dstbuf)

        def scan_body(i, cnt):
            s16 = srcbuf[pl.ds(i * 16, 16)]
            d16 = dstbuf[pl.ds(i * 16, 16)]
            msk = (d16 >= base) & (d16 < base + NPS)
            pk = (s16 << 9) | (d16 - base)
            mi = jnp.where(msk, 1, 0).astype(jnp.int32)
            pos = plsc.cumsum(mi) - 1 + cnt
            plsc.store_scatter(buf, [pos], pk, mask=msk)
            npop = plsc.all_reduce_population_count(msk)
            return cnt + jnp.max(npop)

        cnt_c = lax.fori_loop(0, CHUNK // 16, scan_body, 0)
        buf[pl.ds(cnt_c, 16)] = sent_vec
        cnt8 = (cnt_c + 7) & (-8)
        off = pl.multiple_of(w * EB + total, 8)
        pltpu.sync_copy(buf, bucket_ref.at[pl.ds(off, FLUSH)])
        return total + cnt8

    total = lax.fori_loop(0, E // CHUNK, chunk_body, 0)
    for k in range(4):
        buf[pl.ds(k * 16, 16)] = sent_vec
    off = pl.multiple_of(w * EB + total, 8)
    pltpu.sync_copy(buf.at[pl.ds(0, 64)], bucket_ref.at[pl.ds(off, 64)])
    count_out = (total + 63) & (-64)
    cntb[...] = jnp.full((16,), count_out, dtype=jnp.int32)
    pltpu.sync_copy(cntb, counts_ref.at[pl.ds(pl.multiple_of(w * 16, 8), 16)])


_bucket = pl.kernel(
    _bucket_body,
    compiler_params=pltpu.CompilerParams(needs_layout_passes=False),
    out_type=(
        jax.ShapeDtypeStruct((NW * EB,), jnp.int32),
        jax.ShapeDtypeStruct((NW * 16,), jnp.int32),
    ),
    mesh=_mesh,
    scratch_types=[
        pltpu.VMEM((CHUNK,), jnp.int32),
        pltpu.VMEM((CHUNK,), jnp.int32),
        pltpu.VMEM((FLUSH,), jnp.int32),
        pltpu.VMEM((16,), jnp.int32),
    ],
)


GB = 64


def _segmax_body(p_ref, bucket_ref, counts_ref, out_ref,
                 pkbuf, srcv, dlv, rows, acc, cntb, sem0, sem1):
    w = _wid()
    zeros16 = jnp.zeros((16,), dtype=jnp.float32)

    def zero_body(r, _):
        for f in range(16):
            acc[r, pl.ds(f * 16, 16)] = zeros16
        return 0

    lax.fori_loop(0, ACC_ROWS, zero_body, 0)

    pltpu.sync_copy(counts_ref.at[pl.ds(pl.multiple_of(w * 16, 8), 16)], cntb)
    count = cntb[...][0]
    nb = count >> 6

    def load_pk(b):
        off = pl.multiple_of(w * EB + b * GB, 8)
        pltpu.sync_copy(bucket_ref.at[pl.ds(off, GB)], pkbuf)

    def prep(slot):
        for k in range(4):
            pk = pkbuf[pl.ds(k * 16, 16)]
            srcv[slot, pl.ds(k * 16, 16)] = lax.shift_right_logical(pk, 9)
            dlv[slot, pl.ds(k * 16, 16)] = pk & 511

    def fire(slot):
        @pl.when(slot == 0)
        def _():
            prep(0)
            pltpu.async_copy(p_ref.at[srcv.at[0]], rows.at[0], sem0)

        @pl.when(slot != 0)
        def _():
            prep(1)
            pltpu.async_copy(p_ref.at[srcv.at[1]], rows.at[1], sem1)

    def wait(slot):
        @pl.when(slot == 0)
        def _():
            pltpu.make_async_copy(p_ref.at[srcv.at[0]], rows.at[0],
                                  sem0).wait()

        @pl.when(slot != 0)
        def _():
            pltpu.make_async_copy(p_ref.at[srcv.at[1]], rows.at[1],
                                  sem1).wait()

    @pl.when(nb >= 1)
    def _():
        load_pk(0)
        fire(0)

    def block_body(b, _):
        slot = b & 1

        @pl.when(b + 1 < nb)
        def _():
            load_pk(b + 1)
            fire(slot ^ 1)

        wait(slot)

        def qbody(q, _):
            dl16 = dlv[slot, pl.ds(q * 16, 16)]
            for j in range(16):
                dl = dl16[j]
                qj = q * 16 + j
                for f in range(16):
                    a = acc[dl, pl.ds(f * 16, 16)]
                    r = rows[slot, qj, pl.ds(f * 16, 16)]
                    acc[dl, pl.ds(f * 16, 16)] = jnp.maximum(a, r)
            return 0

        lax.fori_loop(0, GB // 16, qbody, 0)
        return 0

    lax.fori_loop(0, nb, block_body, 0)
    pltpu.sync_copy(acc.at[pl.ds(0, NPS)],
                    out_ref.at[pl.ds(pl.multiple_of(w * NPS, 8), NPS)])


_segmax = pl.kernel(
    _segmax_body,
    out_type=jax.ShapeDtypeStruct((NPAD, D), jnp.float32),
    mesh=_mesh,
    scratch_types=[
        pltpu.VMEM((GB,), jnp.int32),
        pltpu.VMEM((2, GB), jnp.int32),
        pltpu.VMEM((2, GB), jnp.int32),
        pltpu.VMEM((2, GB, D), jnp.float32),
        pltpu.VMEM((ACC_ROWS, D), jnp.float32),
        pltpu.VMEM((16,), jnp.int32),
        pltpu.SemaphoreType.DMA,
        pltpu.SemaphoreType.DMA,
    ],
)


_RB = 1000


def _pool_body(x_ref, wp_ref, bp_ref, ws_ref, b_ref, p_ref, s_ref):
    x = x_ref[...]
    p_ref[...] = jnp.maximum(
        jnp.dot(x, wp_ref[...], preferred_element_type=jnp.float32)
        + bp_ref[...], 0.0)
    s_ref[...] = (jnp.dot(x, ws_ref[...], preferred_element_type=jnp.float32)
                  + b_ref[...])


def _pool(x, wp, bp, ws, b):
    return pl.pallas_call(
        _pool_body,
        grid=(N // _RB,),
        in_specs=[
            pl.BlockSpec((_RB, D), lambda i: (i, 0)),
            pl.BlockSpec((D, D), lambda i: (0, 0)),
            pl.BlockSpec((1, D), lambda i: (0, 0)),
            pl.BlockSpec((D, D), lambda i: (0, 0)),
            pl.BlockSpec((1, D), lambda i: (0, 0)),
        ],
        out_specs=[
            pl.BlockSpec((_RB, D), lambda i: (i, 0)),
            pl.BlockSpec((_RB, D), lambda i: (i, 0)),
        ],
        out_shape=[
            jax.ShapeDtypeStruct((N, D), jnp.float32),
            jax.ShapeDtypeStruct((N, D), jnp.float32),
        ],
    )(x, wp, bp.reshape(1, D), ws, b.reshape(1, D))


def _comb_body(s_ref, g_ref, wn_ref, o_ref, *, act):
    o = s_ref[...] + jnp.dot(g_ref[...], wn_ref[...],
                             preferred_element_type=jnp.float32)
    o_ref[...] = jnp.tanh(o) if act else o


def _comb(s, g, wn, act):
    return pl.pallas_call(
        functools.partial(_comb_body, act=act),
        grid=(N // _RB,),
        in_specs=[
            pl.BlockSpec((_RB, D), lambda i: (i, 0)),
            pl.BlockSpec((_RB, D), lambda i: (i, 0)),
            pl.BlockSpec((D, D), lambda i: (0, 0)),
        ],
        out_specs=pl.BlockSpec((_RB, D), lambda i: (i, 0)),
        out_shape=jax.ShapeDtypeStruct((N, D), jnp.float32),
    )(s, g, wn)


def kernel(x, edge_index,
           W_pool1, b_pool1, W_self1, W_neigh1, b1,
           W_pool2, b_pool2, W_self2, W_neigh2, b2,
           W_pool3, b_pool3, W_self3, W_neigh3, b3):
    bucket, counts = _bucket(edge_index[0], edge_index[1])

    def layer(h, wp, bp, ws, wn, b, act):
        p, s = _pool(h, wp, bp, ws, b)
        g = _segmax(p, bucket, counts)[:N]
        return _comb(s, g, wn, act)

    h = layer(x, W_pool1, b_pool1, W_self1, W_neigh1, b1, True)
    h = layer(h, W_pool2, b_pool2, W_self2, W_neigh2, b2, True)
    h = layer(h, W_pool3, b_pool3, W_self3, W_neigh3, b3, False)
    return h
